# baseline (speedup 1.0000x reference)
import functools

import jax
import jax.numpy as jnp
from jax import lax
from jax.experimental import pallas as pl

_B = 128
_V = 32128
_K = 64
_BLK = 64
_NT = _V // 128


def _body(x_ref, o_ref):
    sign = jnp.int32(-0x80000000)
    low31 = jnp.int32(0x7FFFFFFF)
    x = x_ref[...]

    cm = x[:, 0:128]
    for i in range(1, _NT):
        cm = jnp.maximum(cm, x[:, i * 128:(i + 1) * 128])
    cmb = lax.bitcast_convert_type(cm, jnp.int32)
    cmk = jnp.where(cmb < 0, cmb ^ low31, cmb)

    def mini(i, up):
        bit = lax.shift_left(jnp.int32(1), jnp.int32(31) - i)
        ut = up | bit
        st = ut ^ sign
        c = jnp.sum((cmk >= st).astype(jnp.int32), axis=1, keepdims=True)
        return jnp.where(c >= _K, ut, up)

    ut0 = lax.fori_loop(0, 32, mini, jnp.zeros((_BLK, 1), jnp.int32))

    maxkey = jnp.max(cmk, axis=1, keepdims=True)
    umax = maxkey ^ sign
    mfb = jnp.where(maxkey < 0, maxkey ^ low31, maxkey)
    m = lax.bitcast_convert_type(mfb, jnp.float32)

    d = ut0 ^ umax
    d = d | lax.shift_right_logical(d, 1)
    d = d | lax.shift_right_logical(d, 2)
    d = d | lax.shift_right_logical(d, 4)
    d = d | lax.shift_right_logical(d, 8)
    d = d | lax.shift_right_logical(d, 16)
    duni = jnp.broadcast_to(jnp.max(d | jnp.int32(0xFFFF)), (_BLK, 1))
    prefix0 = umax & ~duni
    bit0 = lax.shift_right_logical(duni, 1) + 1
    pc = duni - (lax.shift_right_logical(duni, 1) & jnp.int32(0x55555555))
    pc = ((pc & jnp.int32(0x33333333)) +
          (lax.shift_right_logical(pc, 2) & jnp.int32(0x33333333)))
    pc = (pc + lax.shift_right_logical(pc, 4)) & jnp.int32(0x0F0F0F0F)
    pc = lax.shift_right_logical(pc * jnp.int32(0x01010101), 24)
    n_it_a = jnp.max(pc) - 16

    def step(i, carry):
        up, bitv = carry
        ut = up | bitv
        st = ut ^ sign
        fb = jnp.where(st < 0, st ^ low31, st)
        tf = lax.bitcast_convert_type(fb, jnp.float32)
        cnt = jnp.sum((x >= tf).astype(jnp.int32), axis=1, keepdims=True)
        up = jnp.where(cnt >= _K, ut, up)
        return (up, lax.shift_right_logical(bitv, 1))

    up, _ = lax.fori_loop(0, n_it_a, step, (prefix0, bit0))

    bits = lax.bitcast_convert_type(x, jnp.int32)
    mkey = jnp.where(bits < 0, bits ^ low31, bits)
    ukey = mkey ^ sign
    uhi = lax.shift_right_logical(ukey, 16)
    p16 = lax.shift_right_logical(up, 16)
    c_above = jnp.sum((uhi > p16).astype(jnp.int32), axis=1, keepdims=True)
    in_bucket = uhi == p16
    lo_b = (ukey & jnp.int32(0xFFFF)) - jnp.int32(32768)
    mlb = jnp.where(in_bucket, lo_b, jnp.int32(-32768)).astype(jnp.int16)

    def step_b(i, carry):
        plo, bitv = carry
        ut = plo | bitv
        stb = (ut - jnp.int32(32768)).astype(jnp.int16)
        c = (mlb >= stb).astype(jnp.int16)
        c = c[:, :16064] + c[:, 16064:]
        c = c[:, :8032] + c[:, 8032:]
        c = c[:, :4016] + c[:, 4016:]
        cnt = jnp.sum(c.astype(jnp.int32), axis=1, keepdims=True) + c_above
        plo = jnp.where(cnt >= _K, ut, plo)
        return (plo, lax.shift_right_logical(bitv, 1))

    plo, _ = lax.fori_loop(0, 16, step_b,
                           (jnp.zeros((_BLK, 1), jnp.int32),
                            jnp.full((_BLK, 1), 0x8000, jnp.int32)))

    ukth = lax.shift_left(p16, 16) | plo
    kkey = ukth ^ sign
    fbits = jnp.where(kkey < 0, kkey ^ low31, kkey)
    kth = lax.bitcast_convert_type(fbits, jnp.float32)

    e = jnp.where(x < kth, 0.0, jnp.exp(x - m))
    z = jnp.sum(e, axis=1, keepdims=True)
    o_ref[...] = e * (1.0 / z)


@jax.jit
def kernel(next_logits, k):
    del k
    return pl.pallas_call(
        _body,
        out_shape=jax.ShapeDtypeStruct((_B, _V), jnp.float32),
        grid=(_B // _BLK,),
        in_specs=[pl.BlockSpec((_BLK, _V), lambda i: (i, 0))],
        out_specs=pl.BlockSpec((_BLK, _V), lambda i: (i, 0)),
    )(next_logits)


# speedup vs baseline: 1.2756x; 1.2756x over previous
import functools

import jax
import jax.numpy as jnp
from jax import lax
from jax.experimental import pallas as pl

_B = 128
_V = 32128
_K = 64
_BLK = 128
_NT = _V // 128


def _body(x_ref, o_ref):
    sign = jnp.int32(-0x80000000)
    low31 = jnp.int32(0x7FFFFFFF)
    x = x_ref[...]

    cm = x[:, 0:128]
    for i in range(1, _NT):
        cm = jnp.maximum(cm, x[:, i * 128:(i + 1) * 128])
    cmb = lax.bitcast_convert_type(cm, jnp.int32)
    cmk = jnp.where(cmb < 0, cmb ^ low31, cmb)

    def mini(i, up):
        bit = lax.shift_left(jnp.int32(1), jnp.int32(31) - i)
        ut = up | bit
        st = ut ^ sign
        c = jnp.sum((cmk >= st).astype(jnp.int32), axis=1, keepdims=True)
        return jnp.where(c >= _K, ut, up)

    ut0 = lax.fori_loop(0, 32, mini, jnp.zeros((_BLK, 1), jnp.int32))

    maxkey = jnp.max(cmk, axis=1, keepdims=True)
    umax = maxkey ^ sign
    mfb = jnp.where(maxkey < 0, maxkey ^ low31, maxkey)
    m = lax.bitcast_convert_type(mfb, jnp.float32)

    d = ut0 ^ umax
    d = d | lax.shift_right_logical(d, 1)
    d = d | lax.shift_right_logical(d, 2)
    d = d | lax.shift_right_logical(d, 4)
    d = d | lax.shift_right_logical(d, 8)
    d = d | lax.shift_right_logical(d, 16)
    prefix0 = umax & ~d
    bit0 = lax.shift_right_logical(d, 1) + 1
    pc = d - (lax.shift_right_logical(d, 1) & jnp.int32(0x55555555))
    pc = ((pc & jnp.int32(0x33333333)) +
          (lax.shift_right_logical(pc, 2) & jnp.int32(0x33333333)))
    pc = (pc + lax.shift_right_logical(pc, 4)) & jnp.int32(0x0F0F0F0F)
    pc = lax.shift_right_logical(pc * jnp.int32(0x01010101), 24)
    n_it = jnp.max(pc)

    def step(i, carry):
        up, bitv = carry
        ut = up | bitv
        st = ut ^ sign
        fb = jnp.where(st < 0, st ^ low31, st)
        tf = lax.bitcast_convert_type(fb, jnp.float32)
        cnt = jnp.sum((x >= tf).astype(jnp.int32), axis=1, keepdims=True)
        up = jnp.where(cnt >= _K, ut, up)
        return (up, lax.shift_right_logical(bitv, 1))

    up, _ = lax.fori_loop(0, n_it, step, (prefix0, bit0))
    kkey = (up ^ sign)
    fbits = jnp.where(kkey < 0, kkey ^ low31, kkey)
    kth = lax.bitcast_convert_type(fbits, jnp.float32)

    e = jnp.where(x < kth, 0.0, jnp.exp(x - m))
    z = jnp.sum(e, axis=1, keepdims=True)
    o_ref[...] = e * (1.0 / z)


@jax.jit
def kernel(next_logits, k):
    del k
    return pl.pallas_call(
        _body,
        out_shape=jax.ShapeDtypeStruct((_B, _V), jnp.float32),
        grid=(_B // _BLK,),
        in_specs=[pl.BlockSpec((_BLK, _V), lambda i: (i, 0))],
        out_specs=pl.BlockSpec((_BLK, _V), lambda i: (i, 0)),
    )(next_logits)
